# Initial kernel scaffold; baseline (speedup 1.0000x reference)
import jax
import jax.numpy as jnp
from jax.experimental import pallas as pl


def kernel(transactions_cat_features, product_feature, tables, product_table):
    raise NotImplementedError("write your pallas kernel here")


# speedup vs baseline: 1.1114x; 1.1114x over previous
import functools

import jax
import jax.numpy as jnp
from jax import lax
from jax.experimental import pallas as pl
from jax.experimental.pallas import tpu as pltpu
from jax.experimental.pallas import tpu_sc as plsc

_NF = 26
_V = 100001
_D = 32
_B = 1024
_L = 50
_NTOK = _B * _L
_NCORES = 2
_NSUB = 16
_NW = _NCORES * _NSUB
_T = 128
_NCHUNK = _NTOK // _T
_NFLD = _NF + 1


def _make_kernel():
    mesh = plsc.VectorSubcoreMesh(core_axis_name="c", subcore_axis_name="s")

    @functools.partial(
        pl.kernel,
        out_type=jax.ShapeDtypeStruct((_NTOK * _NFLD, 1, _D), jnp.float32),
        mesh=mesh,
        scratch_types=[
            pltpu.VMEM((_NFLD, _T), jnp.int32),
            pltpu.VMEM((_NFLD, _T), jnp.int32),
            pltpu.VMEM((_T, 1, _D), jnp.float32),
            pltpu.VMEM((_T, 1, _D), jnp.float32),
            pltpu.SemaphoreType.DMA,
            pltpu.SemaphoreType.DMA,
        ],
    )
    def emb(goff, soff, tables, ptable, out, goff_v, soff_v, rows0, rows1,
            gsem, ssem):
        w = lax.axis_index("c") * _NSUB + lax.axis_index("s")
        nchunks = jnp.where(w < _NCHUNK - 12 * _NW, 13, 12)
        bufs = (rows0, rows1)

        def body(i, carry):
            c = w + i * _NW
            pltpu.sync_copy(goff.at[c], goff_v)
            pltpu.sync_copy(soff.at[c], soff_v)
            gh = [None] * _NFLD
            sh = [None] * _NFLD
            for f in range(_NFLD):
                buf = bufs[f % 2]
                if f >= 2:
                    sh[f - 2].wait()
                src = tables.at[f] if f < _NF else ptable
                gh[f] = pltpu.async_copy(src.at[goff_v.at[f]], buf, gsem)
                if f >= 1:
                    gh[f - 1].wait()
                    sh[f - 1] = pltpu.async_copy(
                        bufs[(f - 1) % 2], out.at[soff_v.at[f - 1]], ssem
                    )
            gh[_NFLD - 1].wait()
            sh[_NFLD - 1] = pltpu.async_copy(
                bufs[(_NFLD - 1) % 2], out.at[soff_v.at[_NFLD - 1]], ssem
            )
            sh[_NFLD - 2].wait()
            sh[_NFLD - 1].wait()
            return carry

        lax.fori_loop(0, nchunks, body, 0)

    return emb


_EMB = _make_kernel()


def kernel(transactions_cat_features, product_feature, tables, product_table):
    trans = transactions_cat_features.astype(jnp.int32)
    g_fields = (trans.reshape(_NF, _NCHUNK, _T) * 4).transpose(1, 0, 2)
    g_prod = (
        jnp.broadcast_to(
            product_feature.astype(jnp.int32)[:, None] * 4, (_B, _L)
        ).reshape(_NCHUNK, 1, _T)
    )
    goff = jnp.concatenate([g_fields, g_prod], axis=1)
    tok = jnp.arange(_NTOK, dtype=jnp.int32).reshape(_NCHUNK, 1, _T)
    fld = jnp.arange(_NFLD, dtype=jnp.int32).reshape(1, _NFLD, 1)
    soff = (tok * _NFLD + fld) * 4
    tables4 = tables.reshape(_NF, _V, 1, _D)
    ptable4 = product_table.reshape(_V, 1, _D)
    out4 = _EMB(goff, soff, tables4, ptable4)
    return out4.reshape(_B, _L, _NFLD * _D)
